# Initial kernel scaffold; baseline (speedup 1.0000x reference)
import jax
import jax.numpy as jnp
from jax.experimental import pallas as pl


def kernel(x, edge_index, edge_attr, batch, W1, b1, Wc0, bc0, Wc1, bc1, Wl1, bl1, Wl2, bl2, Wl3, bl3):
    raise NotImplementedError("write your pallas kernel here")


# speedup vs baseline: 13.7811x; 13.7811x over previous
import functools

import jax
import jax.numpy as jnp
from jax import lax
from jax.experimental import pallas as pl
from jax.experimental.pallas import tpu as pltpu
from jax.experimental.pallas import tpu_sc as plsc

NN = 10000
EE = 320000
DD = 128
NG = 16
NCORES = 2
NSUB = 16
NW = NCORES * NSUB
EPW = EE // NW
EC = 125
CH = EPW // EC
RC = 80
NCHK = NN // RC
F32 = jnp.float32

_MESH = plsc.VectorSubcoreMesh(core_axis_name="c", subcore_axis_name="s")


@functools.partial(
    pl.kernel,
    out_type=jax.ShapeDtypeStruct((NW, NN), F32),
    mesh=_MESH,
    compiler_params=pltpu.CompilerParams(needs_layout_passes=False),
    scratch_types=[
        pltpu.VMEM((EPW,), jnp.int32),
        pltpu.VMEM((EPW,), F32),
        pltpu.VMEM((NN,), F32),
    ],
)
def _deg_sc(dst_hbm, w_hbm, out_hbm, dst_v, w_v, deg_v):
    c = lax.axis_index("c")
    s = lax.axis_index("s")
    wid = c * NSUB + s
    pltpu.sync_copy(dst_hbm.at[wid], dst_v)
    pltpu.sync_copy(w_hbm.at[wid], w_v)

    zero16 = jnp.zeros((16,), F32)

    def zbody(i, carry):
        deg_v[pl.ds(i * 16, 16)] = zero16
        return carry

    lax.fori_loop(0, NN // 16, zbody, 0, unroll=8)

    def abody(i, carry):
        sl = pl.ds(i * 16, 16)
        plsc.addupdate_scatter(deg_v, [dst_v[sl]], w_v[sl])
        return carry

    lax.fori_loop(0, EPW // 16, abody, 0, unroll=4)
    pltpu.sync_copy(deg_v, out_hbm.at[wid])


@functools.partial(
    pl.kernel,
    out_type=jax.ShapeDtypeStruct((NCORES * NN, DD), F32),
    mesh=_MESH,
    compiler_params=pltpu.CompilerParams(needs_layout_passes=False),
    scratch_types=[
        pltpu.VMEM((CH, EC), jnp.int32),
        pltpu.VMEM((CH, EC), jnp.int32),
        pltpu.VMEM((CH, EC), F32),
        pltpu.VMEM((EC, DD), F32),
        pltpu.VMEM_SHARED((NN, DD), F32),
        pltpu.SemaphoreType.DMA,
    ],
)
def _edge_sc(g_hbm, src_hbm, dst_hbm, w_hbm, out_hbm,
             src_v, dst_v, w_v, rows_v, accum_sh, sem):
    c = lax.axis_index("c")
    s = lax.axis_index("s")
    wid = c * NSUB + s
    pltpu.sync_copy(src_hbm.at[wid], src_v)
    pltpu.sync_copy(dst_hbm.at[wid], dst_v)
    pltpu.sync_copy(w_hbm.at[wid], w_v)

    zero16 = jnp.zeros((16,), F32)

    def zbody(i, carry):
        row = rows_v.at[i]
        for k in range(DD // 16):
            row[pl.ds(k * 16, 16)] = zero16
        return carry

    lax.fori_loop(0, RC, zbody, 0)
    for i in range(pl.cdiv(NCHK, NSUB)):
        g = s + i * NSUB

        @pl.when(g < NCHK)
        def _():
            pltpu.sync_copy(rows_v.at[pl.ds(0, RC)],
                            accum_sh.at[pl.ds(g * RC, RC)])

    plsc.subcore_barrier()

    def chunk_body(ci, carry):
        idx = src_v.at[ci]
        pltpu.async_copy(g_hbm.at[idx], rows_v, sem).wait()

        def ebody(e, ecarry):
            wv = plsc.load_gather(
                w_v, [jnp.broadcast_to(ci, (16,)), jnp.broadcast_to(e, (16,))])
            row = rows_v.at[e]
            for k in range(DD // 16):
                sl = pl.ds(k * 16, 16)
                row[sl] = row[sl] * wv
            return ecarry

        lax.fori_loop(0, EC, ebody, 0, unroll=5)
        pltpu.sync_copy(rows_v, accum_sh.at[dst_v.at[ci]], add=True)
        return carry

    lax.fori_loop(0, CH, chunk_body, 0)
    plsc.subcore_barrier()

    for i in range(pl.cdiv(NCHK, NSUB)):
        g = s + i * NSUB

        @pl.when(g < NCHK)
        def _():
            pltpu.sync_copy(accum_sh.at[pl.ds(g * RC, RC)],
                            rows_v.at[pl.ds(0, RC)])
            pltpu.sync_copy(rows_v.at[pl.ds(0, RC)],
                            out_hbm.at[pl.ds(c * NN + g * RC, RC)])


def _tc_prep_body(degp_ref, x_ref, w1_ref, dinv_ref, g1_ref):
    deg = jnp.sum(degp_ref[...], axis=0) + 1.0
    dinv = jnp.where(deg > 0, lax.rsqrt(jnp.maximum(deg, 1e-12)), 0.0)
    dinv_ref[...] = dinv[:, None]
    h1 = jnp.dot(x_ref[...], w1_ref[...], preferred_element_type=F32)
    g1_ref[...] = dinv[:, None] * h1


_tc_prep = pl.pallas_call(
    _tc_prep_body,
    out_shape=(jax.ShapeDtypeStruct((NN, 1), F32),
               jax.ShapeDtypeStruct((NN, DD), F32)),
)


def _tc_mid_body(p_ref, g_ref, dinv_ref, b_ref, w_ref, conv_ref, gnext_ref):
    dinv = dinv_ref[...]
    acc = p_ref[pl.ds(0, NN), :] + p_ref[pl.ds(NN, NN), :] + g_ref[...]
    conv = dinv * acc + b_ref[...]
    conv_ref[...] = conv
    a = jnp.maximum(conv, 0.0)
    gnext_ref[...] = dinv * jnp.dot(a, w_ref[...], preferred_element_type=F32)


_tc_mid = pl.pallas_call(
    _tc_mid_body,
    out_shape=(jax.ShapeDtypeStruct((NN, DD), F32),
               jax.ShapeDtypeStruct((NN, DD), F32)),
)


def _tc_final_body(p_ref, g_ref, dinv_ref, b_ref, batch_ref,
                   wl1_ref, bl1_ref, wl2_ref, bl2_ref, wl3_ref, bl3_ref,
                   conv_ref, pooled_ref, out_ref):
    dinv = dinv_ref[...]
    acc = p_ref[pl.ds(0, NN), :] + p_ref[pl.ds(NN, NN), :] + g_ref[...]
    conv = dinv * acc + b_ref[...]
    conv_ref[...] = conv
    h = jnp.maximum(conv, 0.0)
    seg = lax.broadcasted_iota(jnp.int32, (NG, NN), 0)
    onehot = (seg == batch_ref[...]).astype(F32)
    sums = jnp.dot(onehot, h, preferred_element_type=F32)
    cnt = jnp.sum(onehot, axis=1, keepdims=True)
    pooled = sums / jnp.maximum(cnt, 1.0)
    pooled_ref[...] = pooled
    o = jnp.maximum(jnp.dot(pooled, wl1_ref[...],
                            preferred_element_type=F32) + bl1_ref[...], 0.0)
    o = jnp.maximum(jnp.dot(o, wl2_ref[...],
                            preferred_element_type=F32) + bl2_ref[...], 0.0)
    o = jnp.maximum(jnp.dot(o, wl3_ref[...],
                            preferred_element_type=F32) + bl3_ref[...], 0.0)
    out_ref[...] = o


_tc_final = pl.pallas_call(
    _tc_final_body,
    out_shape=(jax.ShapeDtypeStruct((NN, DD), F32),
               jax.ShapeDtypeStruct((NG, DD), F32),
               jax.ShapeDtypeStruct((NG, 2), F32)),
)


def kernel(x, edge_index, edge_attr, batch,
           W1, b1, Wc0, bc0, Wc1, bc1, Wl1, bl1, Wl2, bl2, Wl3, bl3):
    src = edge_index[0]
    dst = edge_index[1]
    src3 = src.reshape(NW, CH, EC)
    dst3 = dst.reshape(NW, CH, EC)
    w3 = edge_attr.reshape(NW, CH, EC)
    dstf = dst.reshape(NW, EPW)
    wf = edge_attr.reshape(NW, EPW)
    batch2 = batch.reshape(1, NN)
    b1r = b1.reshape(1, DD)
    bc0r = bc0.reshape(1, DD)
    bc1r = bc1.reshape(1, DD)
    bl1r = bl1.reshape(1, -1)
    bl2r = bl2.reshape(1, -1)
    bl3r = bl3.reshape(1, -1)

    degp = _deg_sc(dstf, wf)
    dinv, g1 = _tc_prep(degp, x, W1)
    p = _edge_sc(g1, src3, dst3, w3)
    conv1, g2 = _tc_mid(p, g1, dinv, b1r, Wc0)
    p = _edge_sc(g2, src3, dst3, w3)
    conv2, g3 = _tc_mid(p, g2, dinv, bc0r, Wc1)
    p = _edge_sc(g3, src3, dst3, w3)
    conv3, pooled, out = _tc_final(p, g3, dinv, bc1r, batch2,
                                   Wl1, bl1r, Wl2, bl2r, Wl3, bl3r)
    return (out, pooled, conv1, conv2, conv3)
